# baseline (speedup 1.0000x reference)
import functools

import jax
import jax.numpy as jnp
from jax import lax
from jax.experimental import pallas as pl
from jax.experimental.pallas import tpu as pltpu
from jax.experimental.pallas import tpu_sc as plsc

N_LAYER = 2
A_COEF = 0.1
NEG_BIG = -1e30
NC = 2
NS = 16
EC = 128


def _rup(x, m):
    return (x + m - 1) // m * m


def _sim_body(n_real, ctx, ctxt, out):
    z = jnp.dot(ctx[...], ctxt[...], preferred_element_type=jnp.float32)
    col = lax.broadcasted_iota(jnp.int32, z.shape, 1)
    z = jnp.where(col < n_real, z, NEG_BIG)
    z = z - jnp.max(z, axis=1, keepdims=True)
    p = jnp.exp(z)
    p = jnp.where(col < n_real, p, 0.0)
    sim = p / jnp.sum(p, axis=1, keepdims=True)
    out[...] = sim.astype(jnp.bfloat16)


def _sim_softmax(ctx, n_pad):
    n, d = ctx.shape
    ctx_p = jnp.zeros((n_pad, d), jnp.float32).at[:n].set(ctx)
    rb = 256 if n_pad % 256 == 0 else 128
    return pl.pallas_call(
        functools.partial(_sim_body, n),
        grid=(n_pad // rb,),
        in_specs=[pl.BlockSpec((rb, d), lambda i: (i, 0)),
                  pl.BlockSpec((d, n_pad), lambda i: (0, 0))],
        out_specs=pl.BlockSpec((rb, n_pad), lambda i: (i, 0)),
        out_shape=jax.ShapeDtypeStruct((n_pad, n_pad), jnp.bfloat16),
    )(ctx_p, ctx_p.T)


def _att_body(n_real, nk, kb, pr, r_blk, sim_blk, out, acc):
    k = pl.program_id(1)

    @pl.when(k == 0)
    def _():
        acc[...] = jnp.zeros_like(acc)

    acc[...] += jnp.dot(r_blk[...].astype(jnp.bfloat16), sim_blk[...],
                        preferred_element_type=jnp.float32)

    @pl.when(k == nk - 1)
    def _():
        def leaky_masked(c):
            a = acc[:, pl.ds(c * kb, kb)]
            l = jnp.where(a > 0, a, 0.01 * a)
            col = c * kb + lax.broadcasted_iota(jnp.int32, (pr, kb), 1)
            return jnp.where(col < n_real, l, NEG_BIG)

        def mx_body(c, m):
            return jnp.maximum(m, jnp.max(leaky_masked(c), axis=1,
                                          keepdims=True))

        m = lax.fori_loop(0, nk, mx_body, jnp.full((pr, 1), NEG_BIG,
                                                   jnp.float32))

        def sum_body(c, s):
            p = jnp.exp(leaky_masked(c) - m)
            acc[:, pl.ds(c * kb, kb)] = p
            return s + jnp.sum(p, axis=1, keepdims=True)

        s = lax.fori_loop(0, nk, sum_body, jnp.zeros((pr, 1), jnp.float32))
        inv = 1.0 / s

        def wr_body(c, carry):
            out[:, pl.ds(c * kb, kb)] = (
                acc[:, pl.ds(c * kb, kb)] * inv).astype(jnp.bfloat16)
            return carry

        lax.fori_loop(0, nk, wr_body, 0)


def _att_from_dense(r_dense, sim, n_real):
    rp, kp = r_dense.shape
    pr = 512 if rp % 512 == 0 else 128
    kb = 256 if kp % 256 == 0 else 128
    nk = kp // kb
    return pl.pallas_call(
        functools.partial(_att_body, n_real, nk, kb, pr),
        grid=(rp // pr, nk),
        in_specs=[pl.BlockSpec((pr, kb), lambda i, k: (i, k)),
                  pl.BlockSpec((kb, kp), lambda i, k: (k, 0))],
        out_specs=pl.BlockSpec((pr, kp), lambda i, k: (i, 0)),
        out_shape=jax.ShapeDtypeStruct((rp, kp), jnp.bfloat16),
        scratch_shapes=[pltpu.VMEM((pr, kp), jnp.float32)],
        compiler_params=pltpu.CompilerParams(
            dimension_semantics=("arbitrary", "arbitrary")),
    )(r_dense, sim)


def _update_body(att_blk, tgt, u_blk, out):
    delta = jnp.dot(att_blk[...], tgt[...], preferred_element_type=jnp.float32)
    x = u_blk[...] + A_COEF * delta
    nrm = jnp.sqrt(jnp.sum(x * x, axis=1, keepdims=True))
    out[...] = x / jnp.maximum(nrm, 1e-12)


def _att_update(att, tgt_pad_bf16, u):
    rp, kp = att.shape
    d = u.shape[1]
    pr = 1024 if rp % 1024 == 0 else 128
    return pl.pallas_call(
        _update_body,
        grid=(rp // pr,),
        in_specs=[pl.BlockSpec((pr, kp), lambda i: (i, 0)),
                  pl.BlockSpec((kp, d), lambda i: (0, 0)),
                  pl.BlockSpec((pr, d), lambda i: (i, 0))],
        out_specs=pl.BlockSpec((pr, d), lambda i: (i, 0)),
        out_shape=jax.ShapeDtypeStruct((rp, d), jnp.float32),
    )(att, tgt_pad_bf16, u)


def _mean_concat_body(a0, a1, a2, b0, b1, b2, out):
    left = (a0[...] + a1[...] + a2[...]) * (1.0 / 3.0)
    right = (b0[...] + b1[...] + b2[...]) * (1.0 / 3.0)
    out[...] = jnp.concatenate([left, right], axis=1)


def _mean_concat(a_parts, b_parts):
    n, d = a_parts[0].shape
    blk = 400 if n % 400 == 0 else n
    return pl.pallas_call(
        _mean_concat_body,
        grid=(n // blk,),
        in_specs=[pl.BlockSpec((blk, d), lambda i: (i, 0))] * 6,
        out_specs=pl.BlockSpec((blk, 2 * d), lambda i: (i, 0)),
        out_shape=jax.ShapeDtypeStruct((n, 2 * d), jnp.float32),
    )(*a_parts, *b_parts)


def _scale_wave(w0, val_v, rows_v, ec, wave):
    def grp(g, carry):
        for k in range(wave):
            v16 = val_v[w0 + k, pl.ds(g * 16, 16)]
            for e in range(16):
                vb = v16.at[jnp.full((16,), e, jnp.int32)].get(
                    mode="promise_in_bounds")
                for d in range(4):
                    rows_v[k, g * 16 + e, pl.ds(d * 16, 16)] = (
                        rows_v[k, g * 16 + e, pl.ds(d * 16, 16)] * vb)
        return carry

    lax.fori_loop(0, ec // 16, grp, 0)


def _spmm_superchunk(jb, ec, sch, wave, row2d, col2d, val2d, x_hbm, acc,
                     ridx_v, cidx_v, val_v, rows_v, sem_i, sem_g, sem_s,
                     remap=None):
    for d in (pltpu.async_copy(row2d.at[pl.ds(jb, sch)], ridx_v, sem_i),
              pltpu.async_copy(col2d.at[pl.ds(jb, sch)], cidx_v, sem_i),
              pltpu.async_copy(val2d.at[pl.ds(jb, sch)], val_v, sem_i)):
        d.wait()
    if remap is not None:
        remap()
    for w0 in range(0, sch, wave):
        gathers = [pltpu.async_copy(x_hbm.at[cidx_v.at[w0 + k]],
                                    rows_v.at[k], sem_g)
                   for k in range(wave)]
        for d in gathers:
            d.wait()
        _scale_wave(w0, val_v, rows_v, ec, wave)
        scats = [pltpu.async_copy(rows_v.at[k], acc.at[ridx_v.at[w0 + k]],
                                  sem_s, add=True)
                 for k in range(wave)]
        for d in scats:
            d.wait()


def _sc_spmm_es_body(ept, ec, sch, wave, nrp, rpt, row2d, col2d, val2d,
                     x_hbm, z_hbm, out_hbm, acc, ridx_v, cidx_v, val_v,
                     rows_v, sem_i, sem_g, sem_s):
    c = lax.axis_index("c")
    s = lax.axis_index("s")
    w = c * NS + s
    pltpu.sync_copy(z_hbm.at[pl.ds(s * rpt, rpt)], acc.at[pl.ds(s * rpt, rpt)])
    plsc.subcore_barrier()

    def chunk_body(j, carry):
        jb = pl.multiple_of((w * ept) // ec + j * sch, 8)
        _spmm_superchunk(jb, ec, sch, wave, row2d, col2d, val2d, x_hbm, acc,
                         ridx_v, cidx_v, val_v, rows_v, sem_i, sem_g, sem_s)
        return carry

    lax.fori_loop(0, ept // (ec * sch), chunk_body, 0)
    plsc.subcore_barrier()
    pltpu.sync_copy(acc.at[pl.ds(s * rpt, rpt)],
                    out_hbm.at[pl.ds(c * nrp + s * rpt, rpt)])


def _sc_spmm_rs_body(ept, ec, sch, wave, hp, rpt, row2d, col2d, val2d,
                     x_hbm, z_hbm, out_hbm, acc, ridx_v, cidx_v, val_v,
                     rows_v, sem_i, sem_g, sem_s):
    c = lax.axis_index("c")
    s = lax.axis_index("s")
    pltpu.sync_copy(z_hbm.at[pl.ds(s * rpt, rpt)], acc.at[pl.ds(s * rpt, rpt)])

    @pl.when(s == 0)
    def _():
        pltpu.sync_copy(z_hbm.at[pl.ds(0, 16)], acc.at[pl.ds(hp, 16)])

    plsc.subcore_barrier()

    def remap():
        def rg(g, carry2):
            for k in range(sch):
                r16 = ridx_v[k, pl.ds(g * 16, 16)]
                li = r16 - c * hp
                ok = (li >= 0) & (li < hp)
                ridx_v[k, pl.ds(g * 16, 16)] = jnp.where(ok, li, hp)
            return carry2

        lax.fori_loop(0, ec // 16, rg, 0)

    def chunk_body(j, carry):
        jb = pl.multiple_of((s * ept) // ec + j * sch, 8)
        _spmm_superchunk(jb, ec, sch, wave, row2d, col2d, val2d, x_hbm, acc,
                         ridx_v, cidx_v, val_v, rows_v, sem_i, sem_g, sem_s,
                         remap=remap)
        return carry

    lax.fori_loop(0, ept // (ec * sch), chunk_body, 0)
    plsc.subcore_barrier()
    pltpu.sync_copy(acc.at[pl.ds(s * rpt, rpt)],
                    out_hbm.at[pl.ds(c * hp + s * rpt, rpt)])


def _sc_spmm(row, col, val, x, nrp):
    e = row.shape[0]
    mesh = plsc.VectorSubcoreMesh(core_axis_name="c", subcore_axis_name="s")
    edge_split = nrp <= 12544
    if edge_split:
        ec, sch, wave = 128, 8, 1
        nsplit = NC * NS
    else:
        ec, sch, wave = 128, 8, 1
        nsplit = NS
    epad = _rup(e, nsplit * ec * sch)
    pad = epad - e
    if pad:
        row = jnp.concatenate([row, jnp.zeros((pad,), row.dtype)])
        col = jnp.concatenate([col, jnp.zeros((pad,), col.dtype)])
        val = jnp.concatenate([val, jnp.zeros((pad,), val.dtype)])
    row2d = row.astype(jnp.int32).reshape(epad // ec, ec)
    col2d = col.astype(jnp.int32).reshape(epad // ec, ec)
    val2d = val.reshape(epad // ec, ec)
    ept = epad // nsplit
    if edge_split:
        rpt = nrp // NS
        acc_rows = nrp
        body = functools.partial(_sc_spmm_es_body, ept, ec, sch, wave,
                                 nrp, rpt)
        out_rows = 2 * nrp
    else:
        hp = nrp // 2
        rpt = hp // NS
        acc_rows = hp + 16
        body = functools.partial(_sc_spmm_rs_body, ept, ec, sch, wave,
                                 hp, rpt)
        out_rows = nrp
    k = functools.partial(
        pl.kernel,
        out_type=jax.ShapeDtypeStruct((out_rows, 128), jnp.float32),
        mesh=mesh,
        scratch_types=[
            pltpu.VMEM_SHARED((acc_rows, 128), jnp.float32),
            pltpu.VMEM((sch, ec), jnp.int32),
            pltpu.VMEM((sch, ec), jnp.int32),
            pltpu.VMEM((sch, ec), jnp.float32),
            pltpu.VMEM((wave, ec, 128), jnp.float32),
            pltpu.SemaphoreType.DMA,
            pltpu.SemaphoreType.DMA,
            pltpu.SemaphoreType.DMA,
        ],
    )(body)
    zeros = jnp.zeros((acc_rows, 128), jnp.float32)
    out = k(row2d, col2d, val2d, x, zeros)
    if edge_split:
        return _combine_planes(out, nrp)
    return out


def _add2_body(a, b, out):
    out[...] = a[...] + b[...]


def _combine_planes(planes, nrp):
    d = planes.shape[1]
    blk = 256
    return pl.pallas_call(
        _add2_body,
        grid=(nrp // blk,),
        in_specs=[pl.BlockSpec((blk, d), lambda i: (i, 0)),
                  pl.BlockSpec((blk, d), lambda i: (i, 0))],
        out_specs=pl.BlockSpec((blk, d), lambda i: (i, 0)),
        out_shape=jax.ShapeDtypeStruct((nrp, d), jnp.float32),
    )(lax.slice(planes, (0, 0), (nrp, d)),
      lax.slice(planes, (nrp, 0), (2 * nrp, d)))


def _spmm_sc(row, col, val, x_parts, n_rows):
    nrp = _rup(n_rows, 512)
    x = jnp.concatenate(x_parts, axis=0)
    x = jnp.pad(x, ((0, nrp - x.shape[0]), (0, 128 - x.shape[1])))
    return _sc_spmm(row, col, val, x, nrp)[:n_rows, :64]


def _sc_gather_body(tabs, bpw, u_hbm, p_hbm, n_hbm, au_hbm, ag_hbm, uw_hbm,
                    gw_hbm, o0, o1, o2, o3, o4, o5, idx_v, r128, sem):
    c = lax.axis_index("c")
    s = lax.axis_index("s")
    base = (c * NS + s) * bpw
    outs = (o0, o1, o2, o3, o4, o5)
    srcs = (au_hbm, ag_hbm, ag_hbm, uw_hbm, gw_hbm, gw_hbm)
    idxs = (u_hbm, p_hbm, n_hbm, u_hbm, p_hbm, n_hbm)
    del tabs
    for t in range(6):
        pltpu.sync_copy(idxs[t].at[pl.ds(base, bpw)], idx_v)
        pltpu.async_copy(srcs[t].at[idx_v], r128, sem).wait()
        pltpu.sync_copy(r128, outs[t].at[pl.ds(base, bpw)])


def _sc_gather_outputs(u_idx, p_idx, n_idx, all_users, all_groups,
                       user_w, group_w):
    b = u_idx.shape[0]
    bpw = b // (NC * NS)
    tabs = (128, 128, 128, 64, 64, 64)
    ot = [jax.ShapeDtypeStruct((b, 128), jnp.float32) for _ in tabs]
    k = functools.partial(
        pl.kernel,
        out_type=tuple(ot),
        mesh=plsc.VectorSubcoreMesh(core_axis_name="c", subcore_axis_name="s"),
        scratch_types=[
            pltpu.VMEM((bpw,), jnp.int32),
            pltpu.VMEM((bpw, 128), jnp.float32),
            pltpu.SemaphoreType.DMA,
        ],
    )(functools.partial(_sc_gather_body, tabs, bpw))
    uw_p = jnp.pad(user_w, ((0, 0), (0, 64)))
    gw_p = jnp.pad(group_w, ((0, 0), (0, 64)))
    outs = k(u_idx.astype(jnp.int32), p_idx.astype(jnp.int32),
             n_idx.astype(jnp.int32), all_users, all_groups, uw_p, gw_p)
    return (outs[0], outs[1], outs[2],
            outs[3][:, :64], outs[4][:, :64], outs[5][:, :64])


def _pad_rows(x, n_pad, dtype=None):
    out = jnp.zeros((n_pad, x.shape[1]), x.dtype if dtype is None else dtype)
    return out.at[:x.shape[0]].set(x.astype(out.dtype))


def kernel(user_inputs, pos_groups, neg_groups, user_w, item_w, group_w,
           ctx_item_w, ctx_group_w, adj_row, adj_col, adj_val,
           adj_ui_row, adj_ui_col, adj_ui_val, adj_gi_row, adj_gi_col,
           adj_gi_val, r_row, r_col, r_val, r_ui_row, r_ui_col, r_ui_val):
    n_user = user_w.shape[0]
    n_item = item_w.shape[0]
    n_group = group_w.shape[0]
    base = 1024 if n_user >= 1024 else 128
    up = _rup(n_user, base)
    gp = _rup(n_group, 256)
    ip = _rup(n_item, 256)

    sim_g = _sim_softmax(ctx_group_w, gp)
    sim_i = _sim_softmax(ctx_item_w, ip)

    r_g_dense = jnp.zeros((up, gp), jnp.float32).at[r_row, r_col].add(r_val)
    r_i_dense = jnp.zeros((up, ip), jnp.float32).at[r_ui_row, r_ui_col].add(
        r_ui_val)
    att_g = _att_from_dense(r_g_dense, sim_g, n_group)
    att_i = _att_from_dense(r_i_dense, sim_i, n_item)

    g_u = group_w
    g_i = group_w
    u_g = user_w
    u_i = user_w
    i_u = item_w
    i_g = item_w
    u_g_parts = [u_g]
    u_i_parts = [u_i]
    g_u_parts = [g_u]
    g_i_parts = [g_i]
    u_g_p = _pad_rows(user_w, up)
    u_i_p = u_g_p
    for _ in range(N_LAYER):
        u_g_p = _att_update(att_g, _pad_rows(g_u, gp, jnp.bfloat16), u_g_p)
        u_i_p = _att_update(att_i, _pad_rows(i_u, ip, jnp.bfloat16), u_i_p)
        ug = _spmm_sc(adj_row, adj_col, adj_val,
                      [u_g_p[:n_user], g_u], n_user + n_group)
        u_g, g_u = ug[:n_user], ug[n_user:]
        u_g_parts.append(u_g)
        g_u_parts.append(g_u)
        ui = _spmm_sc(adj_ui_row, adj_ui_col, adj_ui_val,
                      [u_i_p[:n_user], i_u], n_user + n_item)
        u_i, i_u = ui[:n_user], ui[n_user:]
        u_i_parts.append(u_i)
        gi = _spmm_sc(adj_gi_row, adj_gi_col, adj_gi_val,
                      [g_i, i_g], n_group + n_item)
        g_i, i_g = gi[:n_group], gi[n_group:]
        g_i_parts.append(g_i)
        u_g_p = _pad_rows(u_g, up)
        u_i_p = _pad_rows(u_i, up)

    all_users = _mean_concat(u_g_parts, u_i_parts)
    all_groups = _mean_concat(g_u_parts, g_i_parts)
    return _sc_gather_outputs(user_inputs, pos_groups, neg_groups,
                              all_users, all_groups, user_w, group_w)


# speedup vs baseline: 1.2938x; 1.2938x over previous
import functools

import jax
import jax.numpy as jnp
from jax import lax
from jax.experimental import pallas as pl
from jax.experimental.pallas import tpu as pltpu
from jax.experimental.pallas import tpu_sc as plsc

N_LAYER = 2
A_COEF = 0.1
NEG_BIG = -1e30
NC = 2
NS = 16
EC = 128


def _rup(x, m):
    return (x + m - 1) // m * m


def _sim_body(n_real, ctx, ctxt, out):
    z = jnp.dot(ctx[...], ctxt[...], preferred_element_type=jnp.float32)
    col = lax.broadcasted_iota(jnp.int32, z.shape, 1)
    z = jnp.where(col < n_real, z, NEG_BIG)
    z = z - jnp.max(z, axis=1, keepdims=True)
    p = jnp.exp(z)
    p = jnp.where(col < n_real, p, 0.0)
    sim = p / jnp.sum(p, axis=1, keepdims=True)
    out[...] = sim.astype(jnp.bfloat16)


def _sim_softmax(ctx, n_pad):
    n, d = ctx.shape
    ctx_p = jnp.zeros((n_pad, d), jnp.float32).at[:n].set(ctx)
    rb = 256 if n_pad % 256 == 0 else 128
    return pl.pallas_call(
        functools.partial(_sim_body, n),
        grid=(n_pad // rb,),
        in_specs=[pl.BlockSpec((rb, d), lambda i: (i, 0)),
                  pl.BlockSpec((d, n_pad), lambda i: (0, 0))],
        out_specs=pl.BlockSpec((rb, n_pad), lambda i: (i, 0)),
        out_shape=jax.ShapeDtypeStruct((n_pad, n_pad), jnp.bfloat16),
    )(ctx_p, ctx_p.T)


def _att_body(n_real, nk, kb, pr, r_blk, sim_blk, out, acc):
    k = pl.program_id(1)

    @pl.when(k == 0)
    def _():
        acc[...] = jnp.zeros_like(acc)

    acc[...] += jnp.dot(r_blk[...].astype(jnp.bfloat16), sim_blk[...],
                        preferred_element_type=jnp.float32)

    @pl.when(k == nk - 1)
    def _():
        def leaky_masked(c):
            a = acc[:, pl.ds(c * kb, kb)]
            l = jnp.where(a > 0, a, 0.01 * a)
            col = c * kb + lax.broadcasted_iota(jnp.int32, (pr, kb), 1)
            return jnp.where(col < n_real, l, NEG_BIG)

        def mx_body(c, m):
            return jnp.maximum(m, jnp.max(leaky_masked(c), axis=1,
                                          keepdims=True))

        m = lax.fori_loop(0, nk, mx_body, jnp.full((pr, 1), NEG_BIG,
                                                   jnp.float32))

        def sum_body(c, s):
            p = jnp.exp(leaky_masked(c) - m)
            acc[:, pl.ds(c * kb, kb)] = p
            return s + jnp.sum(p, axis=1, keepdims=True)

        s = lax.fori_loop(0, nk, sum_body, jnp.zeros((pr, 1), jnp.float32))
        inv = 1.0 / s

        def wr_body(c, carry):
            out[:, pl.ds(c * kb, kb)] = (
                acc[:, pl.ds(c * kb, kb)] * inv).astype(jnp.bfloat16)
            return carry

        lax.fori_loop(0, nk, wr_body, 0)


def _att_from_dense(r_dense, sim, n_real):
    rp, kp = r_dense.shape
    pr = 512 if rp % 512 == 0 else 128
    kb = 256 if kp % 256 == 0 else 128
    nk = kp // kb
    return pl.pallas_call(
        functools.partial(_att_body, n_real, nk, kb, pr),
        grid=(rp // pr, nk),
        in_specs=[pl.BlockSpec((pr, kb), lambda i, k: (i, k)),
                  pl.BlockSpec((kb, kp), lambda i, k: (k, 0))],
        out_specs=pl.BlockSpec((pr, kp), lambda i, k: (i, 0)),
        out_shape=jax.ShapeDtypeStruct((rp, kp), jnp.bfloat16),
        scratch_shapes=[pltpu.VMEM((pr, kp), jnp.float32)],
        compiler_params=pltpu.CompilerParams(
            dimension_semantics=("arbitrary", "arbitrary")),
    )(r_dense, sim)


def _update_body(att_blk, tgt, u_blk, out):
    delta = jnp.dot(att_blk[...], tgt[...], preferred_element_type=jnp.float32)
    x = u_blk[...] + A_COEF * delta
    nrm = jnp.sqrt(jnp.sum(x * x, axis=1, keepdims=True))
    out[...] = x / jnp.maximum(nrm, 1e-12)


def _att_update(att, tgt_pad_bf16, u):
    rp, kp = att.shape
    d = u.shape[1]
    pr = 1024 if rp % 1024 == 0 else 128
    return pl.pallas_call(
        _update_body,
        grid=(rp // pr,),
        in_specs=[pl.BlockSpec((pr, kp), lambda i: (i, 0)),
                  pl.BlockSpec((kp, d), lambda i: (0, 0)),
                  pl.BlockSpec((pr, d), lambda i: (i, 0))],
        out_specs=pl.BlockSpec((pr, d), lambda i: (i, 0)),
        out_shape=jax.ShapeDtypeStruct((rp, d), jnp.float32),
    )(att, tgt_pad_bf16, u)


def _mean_concat_body(a0, a1, a2, b0, b1, b2, out):
    left = (a0[...] + a1[...] + a2[...]) * (1.0 / 3.0)
    right = (b0[...] + b1[...] + b2[...]) * (1.0 / 3.0)
    out[...] = jnp.concatenate([left, right], axis=1)


def _mean_concat(a_parts, b_parts):
    n, d = a_parts[0].shape
    blk = 400 if n % 400 == 0 else n
    return pl.pallas_call(
        _mean_concat_body,
        grid=(n // blk,),
        in_specs=[pl.BlockSpec((blk, d), lambda i: (i, 0))] * 6,
        out_specs=pl.BlockSpec((blk, 2 * d), lambda i: (i, 0)),
        out_shape=jax.ShapeDtypeStruct((n, 2 * d), jnp.float32),
    )(*a_parts, *b_parts)


def _scale_rows(val_v, rows_v):
    def grp(g, carry2):
        v16 = val_v[pl.ds(g * 16, 16)]
        for e in range(16):
            vb = v16.at[jnp.full((16,), e, jnp.int32)].get(
                mode="promise_in_bounds")
            for d in range(4):
                rows_v[g * 16 + e, pl.ds(d * 16, 16)] = (
                    rows_v[g * 16 + e, pl.ds(d * 16, 16)] * vb)
        return carry2

    lax.fori_loop(0, EC // 16, grp, 0)


def _sc_spmm_es_body(ept, nrp, rpt, row_hbm, col_hbm, val_hbm, x_hbm, z_hbm,
                     out_hbm, acc, ridx_v, cidx_v, val_v, rows_v, sem):
    c = lax.axis_index("c")
    s = lax.axis_index("s")
    w = c * NS + s
    pltpu.sync_copy(z_hbm.at[pl.ds(s * rpt, rpt)], acc.at[pl.ds(s * rpt, rpt)])
    plsc.subcore_barrier()

    def chunk_body(j, carry):
        base = w * ept + j * EC
        pltpu.sync_copy(row_hbm.at[pl.ds(base, EC)], ridx_v)
        pltpu.sync_copy(col_hbm.at[pl.ds(base, EC)], cidx_v)
        pltpu.sync_copy(val_hbm.at[pl.ds(base, EC)], val_v)
        pltpu.async_copy(x_hbm.at[cidx_v], rows_v, sem).wait()
        _scale_rows(val_v, rows_v)
        pltpu.sync_copy(rows_v, acc.at[ridx_v], add=True)
        return carry

    lax.fori_loop(0, ept // EC, chunk_body, 0)
    plsc.subcore_barrier()
    pltpu.sync_copy(acc.at[pl.ds(s * rpt, rpt)],
                    out_hbm.at[pl.ds(c * nrp + s * rpt, rpt)])


def _sc_spmm_rs_body(ept, hp, rpt, row_hbm, col_hbm, val_hbm, x_hbm, z_hbm,
                     out_hbm, acc, ridx_v, cidx_v, val_v, rows_v, sem):
    c = lax.axis_index("c")
    s = lax.axis_index("s")
    pltpu.sync_copy(z_hbm.at[pl.ds(s * rpt, rpt)], acc.at[pl.ds(s * rpt, rpt)])

    @pl.when(s == 0)
    def _():
        pltpu.sync_copy(z_hbm.at[pl.ds(0, 16)], acc.at[pl.ds(hp, 16)])

    plsc.subcore_barrier()

    def chunk_body(j, carry):
        base = s * ept + j * EC
        pltpu.sync_copy(row_hbm.at[pl.ds(base, EC)], ridx_v)
        pltpu.sync_copy(col_hbm.at[pl.ds(base, EC)], cidx_v)
        pltpu.sync_copy(val_hbm.at[pl.ds(base, EC)], val_v)

        def rg(g, carry2):
            r16 = ridx_v[pl.ds(g * 16, 16)]
            li = r16 - c * hp
            ok = (li >= 0) & (li < hp)
            ridx_v[pl.ds(g * 16, 16)] = jnp.where(ok, li, hp)
            return carry2

        lax.fori_loop(0, EC // 16, rg, 0)
        pltpu.async_copy(x_hbm.at[cidx_v], rows_v, sem).wait()
        _scale_rows(val_v, rows_v)
        pltpu.sync_copy(rows_v, acc.at[ridx_v], add=True)
        return carry

    lax.fori_loop(0, ept // EC, chunk_body, 0)
    plsc.subcore_barrier()
    pltpu.sync_copy(acc.at[pl.ds(s * rpt, rpt)],
                    out_hbm.at[pl.ds(c * hp + s * rpt, rpt)])


def _sc_spmm(row, col, val, x, nrp):
    e = row.shape[0]
    mesh = plsc.VectorSubcoreMesh(core_axis_name="c", subcore_axis_name="s")
    edge_split = nrp <= 12544
    nsplit = NC * NS if edge_split else NS
    epad = _rup(e, nsplit * EC)
    pad = epad - e
    if pad:
        row = jnp.concatenate([row, jnp.zeros((pad,), row.dtype)])
        col = jnp.concatenate([col, jnp.zeros((pad,), col.dtype)])
        val = jnp.concatenate([val, jnp.zeros((pad,), val.dtype)])
    row = row.astype(jnp.int32)
    col = col.astype(jnp.int32)
    ept = epad // nsplit
    if edge_split:
        rpt = nrp // NS
        acc_rows = nrp
        body = functools.partial(_sc_spmm_es_body, ept, nrp, rpt)
        out_rows = 2 * nrp
    else:
        hp = nrp // 2
        rpt = hp // NS
        acc_rows = hp + 16
        body = functools.partial(_sc_spmm_rs_body, ept, hp, rpt)
        out_rows = nrp
    k = functools.partial(
        pl.kernel,
        out_type=jax.ShapeDtypeStruct((out_rows, 128), jnp.float32),
        mesh=mesh,
        scratch_types=[
            pltpu.VMEM_SHARED((acc_rows, 128), jnp.float32),
            pltpu.VMEM((EC,), jnp.int32),
            pltpu.VMEM((EC,), jnp.int32),
            pltpu.VMEM((EC,), jnp.float32),
            pltpu.VMEM((EC, 128), jnp.float32),
            pltpu.SemaphoreType.DMA,
        ],
    )(body)
    zeros = jnp.zeros((acc_rows, 128), jnp.float32)
    out = k(row, col, val, x, zeros)
    if edge_split:
        return _combine_planes(out, nrp)
    return out


def _add2_body(a, b, out):
    out[...] = a[...] + b[...]


def _combine_planes(planes, nrp):
    d = planes.shape[1]
    blk = 256
    return pl.pallas_call(
        _add2_body,
        grid=(nrp // blk,),
        in_specs=[pl.BlockSpec((blk, d), lambda i: (i, 0)),
                  pl.BlockSpec((blk, d), lambda i: (i, 0))],
        out_specs=pl.BlockSpec((blk, d), lambda i: (i, 0)),
        out_shape=jax.ShapeDtypeStruct((nrp, d), jnp.float32),
    )(lax.slice(planes, (0, 0), (nrp, d)),
      lax.slice(planes, (nrp, 0), (2 * nrp, d)))


def _spmm_sc(row, col, val, x_parts, n_rows):
    nrp = _rup(n_rows, 512)
    x = jnp.concatenate(x_parts, axis=0)
    x = jnp.pad(x, ((0, nrp - x.shape[0]), (0, 128 - x.shape[1])))
    return _sc_spmm(row, col, val, x, nrp)[:n_rows, :64]


def _sc_gather_body(tabs, bpw, u_hbm, p_hbm, n_hbm, au_hbm, ag_hbm, uw_hbm,
                    gw_hbm, o0, o1, o2, o3, o4, o5, idx_v, r128, sem):
    c = lax.axis_index("c")
    s = lax.axis_index("s")
    base = (c * NS + s) * bpw
    outs = (o0, o1, o2, o3, o4, o5)
    srcs = (au_hbm, ag_hbm, ag_hbm, uw_hbm, gw_hbm, gw_hbm)
    idxs = (u_hbm, p_hbm, n_hbm, u_hbm, p_hbm, n_hbm)
    del tabs
    for t in range(6):
        pltpu.sync_copy(idxs[t].at[pl.ds(base, bpw)], idx_v)
        pltpu.async_copy(srcs[t].at[idx_v], r128, sem).wait()
        pltpu.sync_copy(r128, outs[t].at[pl.ds(base, bpw)])


def _sc_gather_outputs(u_idx, p_idx, n_idx, all_users, all_groups,
                       user_w, group_w):
    b = u_idx.shape[0]
    bpw = b // (NC * NS)
    tabs = (128, 128, 128, 64, 64, 64)
    ot = [jax.ShapeDtypeStruct((b, 128), jnp.float32) for _ in tabs]
    k = functools.partial(
        pl.kernel,
        out_type=tuple(ot),
        mesh=plsc.VectorSubcoreMesh(core_axis_name="c", subcore_axis_name="s"),
        scratch_types=[
            pltpu.VMEM((bpw,), jnp.int32),
            pltpu.VMEM((bpw, 128), jnp.float32),
            pltpu.SemaphoreType.DMA,
        ],
    )(functools.partial(_sc_gather_body, tabs, bpw))
    uw_p = jnp.pad(user_w, ((0, 0), (0, 64)))
    gw_p = jnp.pad(group_w, ((0, 0), (0, 64)))
    outs = k(u_idx.astype(jnp.int32), p_idx.astype(jnp.int32),
             n_idx.astype(jnp.int32), all_users, all_groups, uw_p, gw_p)
    return (outs[0], outs[1], outs[2],
            outs[3][:, :64], outs[4][:, :64], outs[5][:, :64])


def _pad_rows(x, n_pad, dtype=None):
    out = jnp.zeros((n_pad, x.shape[1]), x.dtype if dtype is None else dtype)
    return out.at[:x.shape[0]].set(x.astype(out.dtype))


def kernel(user_inputs, pos_groups, neg_groups, user_w, item_w, group_w,
           ctx_item_w, ctx_group_w, adj_row, adj_col, adj_val,
           adj_ui_row, adj_ui_col, adj_ui_val, adj_gi_row, adj_gi_col,
           adj_gi_val, r_row, r_col, r_val, r_ui_row, r_ui_col, r_ui_val):
    n_user = user_w.shape[0]
    n_item = item_w.shape[0]
    n_group = group_w.shape[0]
    base = 1024 if n_user >= 1024 else 128
    up = _rup(n_user, base)
    gp = _rup(n_group, 256)
    ip = _rup(n_item, 256)

    sim_g = _sim_softmax(ctx_group_w, gp)
    sim_i = _sim_softmax(ctx_item_w, ip)

    r_g_dense = jnp.zeros((up, gp), jnp.float32).at[r_row, r_col].add(r_val)
    r_i_dense = jnp.zeros((up, ip), jnp.float32).at[r_ui_row, r_ui_col].add(
        r_ui_val)
    att_g = _att_from_dense(r_g_dense, sim_g, n_group)
    att_i = _att_from_dense(r_i_dense, sim_i, n_item)

    g_u = group_w
    g_i = group_w
    u_g = user_w
    u_i = user_w
    i_u = item_w
    i_g = item_w
    u_g_parts = [u_g]
    u_i_parts = [u_i]
    g_u_parts = [g_u]
    g_i_parts = [g_i]
    u_g_p = _pad_rows(user_w, up)
    u_i_p = u_g_p
    for _ in range(N_LAYER):
        u_g_p = _att_update(att_g, _pad_rows(g_u, gp, jnp.bfloat16), u_g_p)
        u_i_p = _att_update(att_i, _pad_rows(i_u, ip, jnp.bfloat16), u_i_p)
        ug = _spmm_sc(adj_row, adj_col, adj_val,
                      [u_g_p[:n_user], g_u], n_user + n_group)
        u_g, g_u = ug[:n_user], ug[n_user:]
        u_g_parts.append(u_g)
        g_u_parts.append(g_u)
        ui = _spmm_sc(adj_ui_row, adj_ui_col, adj_ui_val,
                      [u_i_p[:n_user], i_u], n_user + n_item)
        u_i, i_u = ui[:n_user], ui[n_user:]
        u_i_parts.append(u_i)
        gi = _spmm_sc(adj_gi_row, adj_gi_col, adj_gi_val,
                      [g_i, i_g], n_group + n_item)
        g_i, i_g = gi[:n_group], gi[n_group:]
        g_i_parts.append(g_i)
        u_g_p = _pad_rows(u_g, up)
        u_i_p = _pad_rows(u_i, up)

    all_users = _mean_concat(u_g_parts, u_i_parts)
    all_groups = _mean_concat(g_u_parts, g_i_parts)
    return _sc_gather_outputs(user_inputs, pos_groups, neg_groups,
                              all_users, all_groups, user_w, group_w)


# speedup vs baseline: 1.3407x; 1.0363x over previous
import functools

import jax
import jax.numpy as jnp
from jax import lax
from jax.experimental import pallas as pl
from jax.experimental.pallas import tpu as pltpu
from jax.experimental.pallas import tpu_sc as plsc

N_LAYER = 2
A_COEF = 0.1
NEG_BIG = -1e30
NC = 2
NS = 16
EC = 128


def _rup(x, m):
    return (x + m - 1) // m * m


def _sim_body(n_real, ctx, ctxt, out):
    z = jnp.dot(ctx[...], ctxt[...], preferred_element_type=jnp.float32)
    col = lax.broadcasted_iota(jnp.int32, z.shape, 1)
    z = jnp.where(col < n_real, z, NEG_BIG)
    z = z - jnp.max(z, axis=1, keepdims=True)
    p = jnp.exp(z)
    p = jnp.where(col < n_real, p, 0.0)
    sim = p / jnp.sum(p, axis=1, keepdims=True)
    out[...] = sim.astype(jnp.bfloat16)


def _sim_softmax(ctx, n_pad):
    n, d = ctx.shape
    ctx_p = jnp.zeros((n_pad, d), jnp.float32).at[:n].set(ctx)
    rb = 256 if n_pad % 256 == 0 else 128
    return pl.pallas_call(
        functools.partial(_sim_body, n),
        grid=(n_pad // rb,),
        in_specs=[pl.BlockSpec((rb, d), lambda i: (i, 0)),
                  pl.BlockSpec((d, n_pad), lambda i: (0, 0))],
        out_specs=pl.BlockSpec((rb, n_pad), lambda i: (i, 0)),
        out_shape=jax.ShapeDtypeStruct((n_pad, n_pad), jnp.bfloat16),
    )(ctx_p, ctx_p.T)


def _att_body(n_real, nk, kb, pr, r_blk, sim_blk, out, acc):
    k = pl.program_id(1)

    @pl.when(k == 0)
    def _():
        acc[...] = jnp.zeros_like(acc)

    acc[...] += jnp.dot(r_blk[...].astype(jnp.bfloat16), sim_blk[...],
                        preferred_element_type=jnp.float32)

    @pl.when(k == nk - 1)
    def _():
        def leaky_masked(c):
            a = acc[:, pl.ds(c * kb, kb)]
            l = jnp.where(a > 0, a, 0.01 * a)
            col = c * kb + lax.broadcasted_iota(jnp.int32, (pr, kb), 1)
            return jnp.where(col < n_real, l, NEG_BIG)

        def mx_body(c, m):
            return jnp.maximum(m, jnp.max(leaky_masked(c), axis=1,
                                          keepdims=True))

        m = lax.fori_loop(0, nk, mx_body, jnp.full((pr, 1), NEG_BIG,
                                                   jnp.float32))

        def sum_body(c, s):
            p = jnp.exp(leaky_masked(c) - m)
            acc[:, pl.ds(c * kb, kb)] = p
            return s + jnp.sum(p, axis=1, keepdims=True)

        s = lax.fori_loop(0, nk, sum_body, jnp.zeros((pr, 1), jnp.float32))
        inv = 1.0 / s

        def wr_body(c, carry):
            out[:, pl.ds(c * kb, kb)] = (
                acc[:, pl.ds(c * kb, kb)] * inv).astype(jnp.bfloat16)
            return carry

        lax.fori_loop(0, nk, wr_body, 0)


def _att_from_dense(r_dense, sim, n_real):
    rp, kp = r_dense.shape
    pr = 1024 if rp % 1024 == 0 else 128
    kb = 256 if kp % 256 == 0 else 128
    nk = kp // kb
    return pl.pallas_call(
        functools.partial(_att_body, n_real, nk, kb, pr),
        grid=(rp // pr, nk),
        in_specs=[pl.BlockSpec((pr, kb), lambda i, k: (i, k)),
                  pl.BlockSpec((kb, kp), lambda i, k: (k, 0))],
        out_specs=pl.BlockSpec((pr, kp), lambda i, k: (i, 0)),
        out_shape=jax.ShapeDtypeStruct((rp, kp), jnp.bfloat16),
        scratch_shapes=[pltpu.VMEM((pr, kp), jnp.float32)],
        compiler_params=pltpu.CompilerParams(
            dimension_semantics=("arbitrary", "arbitrary")),
    )(r_dense, sim)


def _update_body(att_blk, tgt, u_blk, out):
    delta = jnp.dot(att_blk[...], tgt[...], preferred_element_type=jnp.float32)
    x = u_blk[...] + A_COEF * delta
    nrm = jnp.sqrt(jnp.sum(x * x, axis=1, keepdims=True))
    out[...] = x / jnp.maximum(nrm, 1e-12)


def _att_update(att, tgt_pad_bf16, u):
    rp, kp = att.shape
    d = u.shape[1]
    pr = 1024 if rp % 1024 == 0 else 128
    return pl.pallas_call(
        _update_body,
        grid=(rp // pr,),
        in_specs=[pl.BlockSpec((pr, kp), lambda i: (i, 0)),
                  pl.BlockSpec((kp, d), lambda i: (0, 0)),
                  pl.BlockSpec((pr, d), lambda i: (i, 0))],
        out_specs=pl.BlockSpec((pr, d), lambda i: (i, 0)),
        out_shape=jax.ShapeDtypeStruct((rp, d), jnp.float32),
    )(att, tgt_pad_bf16, u)


def _mean_concat_body(a0, a1, a2, b0, b1, b2, out):
    left = (a0[...] + a1[...] + a2[...]) * (1.0 / 3.0)
    right = (b0[...] + b1[...] + b2[...]) * (1.0 / 3.0)
    out[...] = jnp.concatenate([left, right], axis=1)


def _mean_concat(a_parts, b_parts):
    n, d = a_parts[0].shape
    blk = 400 if n % 400 == 0 else n
    return pl.pallas_call(
        _mean_concat_body,
        grid=(n // blk,),
        in_specs=[pl.BlockSpec((blk, d), lambda i: (i, 0))] * 6,
        out_specs=pl.BlockSpec((blk, 2 * d), lambda i: (i, 0)),
        out_shape=jax.ShapeDtypeStruct((n, 2 * d), jnp.float32),
    )(*a_parts, *b_parts)


def _scale_rows(val_v, rows_v):
    def grp(g, carry2):
        v16 = val_v[pl.ds(g * 16, 16)]
        for e in range(16):
            vb = v16.at[jnp.full((16,), e, jnp.int32)].get(
                mode="promise_in_bounds")
            for d in range(4):
                rows_v[g * 16 + e, pl.ds(d * 16, 16)] = (
                    rows_v[g * 16 + e, pl.ds(d * 16, 16)] * vb)
        return carry2

    lax.fori_loop(0, EC // 16, grp, 0)


def _sc_spmm_es_body(ept, nrp, rpt, row_hbm, col_hbm, val_hbm, x_hbm, z_hbm,
                     out_hbm, acc, ridx_v, cidx_v, val_v, rows_v, sem):
    c = lax.axis_index("c")
    s = lax.axis_index("s")
    w = c * NS + s
    pltpu.sync_copy(z_hbm.at[pl.ds(s * rpt, rpt)], acc.at[pl.ds(s * rpt, rpt)])
    plsc.subcore_barrier()

    def chunk_body(j, carry):
        base = w * ept + j * EC
        pltpu.sync_copy(row_hbm.at[pl.ds(base, EC)], ridx_v)
        pltpu.sync_copy(col_hbm.at[pl.ds(base, EC)], cidx_v)
        pltpu.sync_copy(val_hbm.at[pl.ds(base, EC)], val_v)
        pltpu.async_copy(x_hbm.at[cidx_v], rows_v, sem).wait()
        _scale_rows(val_v, rows_v)
        pltpu.sync_copy(rows_v, acc.at[ridx_v], add=True)
        return carry

    lax.fori_loop(0, ept // EC, chunk_body, 0)
    plsc.subcore_barrier()
    pltpu.sync_copy(acc.at[pl.ds(s * rpt, rpt)],
                    out_hbm.at[pl.ds(c * nrp + s * rpt, rpt)])


def _sc_spmm_rs_body(ept, hp, rpt, row_hbm, col_hbm, val_hbm, x_hbm, z_hbm,
                     out_hbm, acc, ridx_v, cidx_v, val_v, rows_v, sem):
    c = lax.axis_index("c")
    s = lax.axis_index("s")
    pltpu.sync_copy(z_hbm.at[pl.ds(s * rpt, rpt)], acc.at[pl.ds(s * rpt, rpt)])

    @pl.when(s == 0)
    def _():
        pltpu.sync_copy(z_hbm.at[pl.ds(0, 16)], acc.at[pl.ds(hp, 16)])

    plsc.subcore_barrier()

    def chunk_body(j, carry):
        base = s * ept + j * EC
        pltpu.sync_copy(row_hbm.at[pl.ds(base, EC)], ridx_v)
        pltpu.sync_copy(col_hbm.at[pl.ds(base, EC)], cidx_v)
        pltpu.sync_copy(val_hbm.at[pl.ds(base, EC)], val_v)

        def rg(g, carry2):
            r16 = ridx_v[pl.ds(g * 16, 16)]
            li = r16 - c * hp
            ok = (li >= 0) & (li < hp)
            ridx_v[pl.ds(g * 16, 16)] = jnp.where(ok, li, hp)
            return carry2

        lax.fori_loop(0, EC // 16, rg, 0)
        pltpu.async_copy(x_hbm.at[cidx_v], rows_v, sem).wait()
        _scale_rows(val_v, rows_v)
        pltpu.sync_copy(rows_v, acc.at[ridx_v], add=True)
        return carry

    lax.fori_loop(0, ept // EC, chunk_body, 0)
    plsc.subcore_barrier()
    pltpu.sync_copy(acc.at[pl.ds(s * rpt, rpt)],
                    out_hbm.at[pl.ds(c * hp + s * rpt, rpt)])


def _sc_spmm(row, col, val, x, nrp):
    e = row.shape[0]
    mesh = plsc.VectorSubcoreMesh(core_axis_name="c", subcore_axis_name="s")
    edge_split = nrp <= 12544
    nsplit = NC * NS if edge_split else NS
    epad = _rup(e, nsplit * EC)
    pad = epad - e
    if pad:
        row = jnp.concatenate([row, jnp.zeros((pad,), row.dtype)])
        col = jnp.concatenate([col, jnp.zeros((pad,), col.dtype)])
        val = jnp.concatenate([val, jnp.zeros((pad,), val.dtype)])
    row = row.astype(jnp.int32)
    col = col.astype(jnp.int32)
    ept = epad // nsplit
    if edge_split:
        rpt = nrp // NS
        acc_rows = nrp
        body = functools.partial(_sc_spmm_es_body, ept, nrp, rpt)
        out_rows = 2 * nrp
    else:
        hp = nrp // 2
        rpt = hp // NS
        acc_rows = hp + 16
        body = functools.partial(_sc_spmm_rs_body, ept, hp, rpt)
        out_rows = nrp
    k = functools.partial(
        pl.kernel,
        out_type=jax.ShapeDtypeStruct((out_rows, 128), jnp.float32),
        mesh=mesh,
        scratch_types=[
            pltpu.VMEM_SHARED((acc_rows, 128), jnp.float32),
            pltpu.VMEM((EC,), jnp.int32),
            pltpu.VMEM((EC,), jnp.int32),
            pltpu.VMEM((EC,), jnp.float32),
            pltpu.VMEM((EC, 128), jnp.float32),
            pltpu.SemaphoreType.DMA,
        ],
    )(body)
    zeros = jnp.zeros((acc_rows, 128), jnp.float32)
    out = k(row, col, val, x, zeros)
    if edge_split:
        return _combine_planes(out, nrp)
    return out


def _add2_body(a, b, out):
    out[...] = a[...] + b[...]


def _combine_planes(planes, nrp):
    d = planes.shape[1]
    blk = 256
    return pl.pallas_call(
        _add2_body,
        grid=(nrp // blk,),
        in_specs=[pl.BlockSpec((blk, d), lambda i: (i, 0)),
                  pl.BlockSpec((blk, d), lambda i: (i, 0))],
        out_specs=pl.BlockSpec((blk, d), lambda i: (i, 0)),
        out_shape=jax.ShapeDtypeStruct((nrp, d), jnp.float32),
    )(lax.slice(planes, (0, 0), (nrp, d)),
      lax.slice(planes, (nrp, 0), (2 * nrp, d)))


def _spmm_sc(row, col, val, x_parts, n_rows):
    nrp = _rup(n_rows, 512)
    x = jnp.concatenate(x_parts, axis=0)
    x = jnp.pad(x, ((0, nrp - x.shape[0]), (0, 128 - x.shape[1])))
    return _sc_spmm(row, col, val, x, nrp)[:n_rows, :64]


def _sc_gather_body(tabs, bpw, u_hbm, p_hbm, n_hbm, au_hbm, ag_hbm, uw_hbm,
                    gw_hbm, o0, o1, o2, o3, o4, o5, idx_v, r128, sem):
    c = lax.axis_index("c")
    s = lax.axis_index("s")
    base = (c * NS + s) * bpw
    outs = (o0, o1, o2, o3, o4, o5)
    srcs = (au_hbm, ag_hbm, ag_hbm, uw_hbm, gw_hbm, gw_hbm)
    idxs = (u_hbm, p_hbm, n_hbm, u_hbm, p_hbm, n_hbm)
    del tabs
    for t in range(6):
        pltpu.sync_copy(idxs[t].at[pl.ds(base, bpw)], idx_v)
        pltpu.async_copy(srcs[t].at[idx_v], r128, sem).wait()
        pltpu.sync_copy(r128, outs[t].at[pl.ds(base, bpw)])


def _sc_gather_outputs(u_idx, p_idx, n_idx, all_users, all_groups,
                       user_w, group_w):
    b = u_idx.shape[0]
    bpw = b // (NC * NS)
    tabs = (128, 128, 128, 64, 64, 64)
    ot = [jax.ShapeDtypeStruct((b, 128), jnp.float32) for _ in tabs]
    k = functools.partial(
        pl.kernel,
        out_type=tuple(ot),
        mesh=plsc.VectorSubcoreMesh(core_axis_name="c", subcore_axis_name="s"),
        scratch_types=[
            pltpu.VMEM((bpw,), jnp.int32),
            pltpu.VMEM((bpw, 128), jnp.float32),
            pltpu.SemaphoreType.DMA,
        ],
    )(functools.partial(_sc_gather_body, tabs, bpw))
    uw_p = jnp.pad(user_w, ((0, 0), (0, 64)))
    gw_p = jnp.pad(group_w, ((0, 0), (0, 64)))
    outs = k(u_idx.astype(jnp.int32), p_idx.astype(jnp.int32),
             n_idx.astype(jnp.int32), all_users, all_groups, uw_p, gw_p)
    return (outs[0], outs[1], outs[2],
            outs[3][:, :64], outs[4][:, :64], outs[5][:, :64])


def _pad_rows(x, n_pad, dtype=None):
    out = jnp.zeros((n_pad, x.shape[1]), x.dtype if dtype is None else dtype)
    return out.at[:x.shape[0]].set(x.astype(out.dtype))


def kernel(user_inputs, pos_groups, neg_groups, user_w, item_w, group_w,
           ctx_item_w, ctx_group_w, adj_row, adj_col, adj_val,
           adj_ui_row, adj_ui_col, adj_ui_val, adj_gi_row, adj_gi_col,
           adj_gi_val, r_row, r_col, r_val, r_ui_row, r_ui_col, r_ui_val):
    n_user = user_w.shape[0]
    n_item = item_w.shape[0]
    n_group = group_w.shape[0]
    base = 1024 if n_user >= 1024 else 128
    up = _rup(n_user, base)
    gp = _rup(n_group, 256)
    ip = _rup(n_item, 256)

    sim_g = _sim_softmax(ctx_group_w, gp)
    sim_i = _sim_softmax(ctx_item_w, ip)

    r_g_dense = jnp.zeros((up, gp), jnp.float32).at[r_row, r_col].add(r_val)
    r_i_dense = jnp.zeros((up, ip), jnp.float32).at[r_ui_row, r_ui_col].add(
        r_ui_val)
    att_g = _att_from_dense(r_g_dense, sim_g, n_group)
    att_i = _att_from_dense(r_i_dense, sim_i, n_item)

    g_u = group_w
    g_i = group_w
    u_g = user_w
    u_i = user_w
    i_u = item_w
    i_g = item_w
    u_g_parts = [u_g]
    u_i_parts = [u_i]
    g_u_parts = [g_u]
    g_i_parts = [g_i]
    u_g_p = _pad_rows(user_w, up)
    u_i_p = u_g_p
    for _ in range(N_LAYER):
        u_g_p = _att_update(att_g, _pad_rows(g_u, gp, jnp.bfloat16), u_g_p)
        u_i_p = _att_update(att_i, _pad_rows(i_u, ip, jnp.bfloat16), u_i_p)
        ug = _spmm_sc(adj_row, adj_col, adj_val,
                      [u_g_p[:n_user], g_u], n_user + n_group)
        u_g, g_u = ug[:n_user], ug[n_user:]
        u_g_parts.append(u_g)
        g_u_parts.append(g_u)
        ui = _spmm_sc(adj_ui_row, adj_ui_col, adj_ui_val,
                      [u_i_p[:n_user], i_u], n_user + n_item)
        u_i, i_u = ui[:n_user], ui[n_user:]
        u_i_parts.append(u_i)
        gi = _spmm_sc(adj_gi_row, adj_gi_col, adj_gi_val,
                      [g_i, i_g], n_group + n_item)
        g_i, i_g = gi[:n_group], gi[n_group:]
        g_i_parts.append(g_i)
        u_g_p = _pad_rows(u_g, up)
        u_i_p = _pad_rows(u_i, up)

    all_users = _mean_concat(u_g_parts, u_i_parts)
    all_groups = _mean_concat(g_u_parts, g_i_parts)
    return _sc_gather_outputs(user_inputs, pos_groups, neg_groups,
                              all_users, all_groups, user_w, group_w)
